# baseline (speedup 1.0000x reference)
import functools

import jax
import jax.numpy as jnp
from jax import lax
from jax.experimental import pallas as pl
from jax.experimental.pallas import tpu as pltpu
from jax.experimental.pallas import tpu_sc as plsc

L = 16
NC = 2
NS = 16

N = 10000
E = 160000
D = 256

RPT = 320
HALF = RPT * NS
NPAD = HALF * NC
DUMMY = HALF
ZPT = 321
ACC_ROWS = ZPT * NS

EPT = E // NS
GRP = EPT // L

RGRP = (313, 312)
K = 48
EBUF = RGRP[0] * L + 2 * K

_MESH = plsc.VectorSubcoreMesh(core_axis_name="c", subcore_axis_name="s")
_SC_PARAMS = pltpu.CompilerParams(needs_layout_passes=False,
                                  use_tc_tiling_on_sc=False)


def _rsqrt_newton(d):
    i = lax.bitcast_convert_type(d, jnp.int32)
    i = 0x5F3759DF - lax.shift_right_logical(i, 1)
    y = lax.bitcast_convert_type(i, jnp.float32)
    for _ in range(3):
        y = y * (1.5 - 0.5 * d * y * y)
    return y


HR = NPAD // L


def _degree_norm_body(src_hbm, dst_hbm, ns_out, nd_out,
                      acc_s, acc_d, hist_s, hist_d, ebuf_s, ebuf_d,
                      idx_id, nb):
    c = lax.axis_index("c")
    s = lax.axis_index("s")
    zero16 = jnp.zeros((L,), jnp.float32)
    iota16 = lax.iota(jnp.int32, L)

    def fill(i, _):
        hist_s[i, :] = zero16
        hist_d[i, :] = zero16
        return 0
    lax.fori_loop(0, HR, fill, 0)

    def fill_idx(g, _):
        idx_id[pl.ds(g * L, L)] = iota16 + g * L
        return 0
    lax.fori_loop(0, HR // L, fill_idx, 0)

    seg = HR // NS
    pltpu.sync_copy(hist_s.at[pl.ds(0, seg)], acc_s.at[pl.ds(s * seg, seg)])
    pltpu.sync_copy(hist_s.at[pl.ds(0, seg)], acc_d.at[pl.ds(s * seg, seg)])

    pltpu.sync_copy(src_hbm.at[pl.ds(s * EPT, EPT)], ebuf_s)
    pltpu.sync_copy(dst_hbm.at[pl.ds(s * EPT, EPT)], ebuf_d)
    plsc.subcore_barrier()

    def scan(g, _):
        sv = ebuf_s[pl.ds(g * L, L)]
        dv = ebuf_d[pl.ds(g * L, L)]
        cs, ms = plsc.scan_count(sv)
        cd, md = plsc.scan_count(dv)
        plsc.addupdate_scatter(
            hist_s, [lax.shift_right_logical(sv, 4), sv & (L - 1)],
            cs.astype(jnp.float32), mask=ms)
        plsc.addupdate_scatter(
            hist_d, [lax.shift_right_logical(dv, 4), dv & (L - 1)],
            cd.astype(jnp.float32), mask=md)
        return 0
    lax.fori_loop(0, GRP, scan, 0)

    pltpu.sync_copy(hist_s, acc_s.at[idx_id], add=True)
    pltpu.sync_copy(hist_d, acc_d.at[idx_id], add=True)
    plsc.subcore_barrier()

    nrows = RPT // L
    for acc, out in ((acc_s, ns_out), (acc_d, nd_out)):
        pltpu.sync_copy(acc.at[pl.ds(c * (HALF // L) + s * nrows, nrows)], nb)

        def norm(k, _):
            deg = nb[k, :]
            nb[k, :] = _rsqrt_newton(jnp.maximum(deg, 1.0))
            return 0
        lax.fori_loop(0, nrows, norm, 0)
        pltpu.sync_copy(nb, out.at[pl.ds(c * (HALF // L) + s * nrows, nrows)])


@functools.partial(
    pl.kernel,
    out_type=(jax.ShapeDtypeStruct((HR, L), jnp.float32),
              jax.ShapeDtypeStruct((HR, L), jnp.float32)),
    mesh=_MESH,
    scratch_types=[
        pltpu.VMEM_SHARED((HR, L), jnp.float32),
        pltpu.VMEM_SHARED((HR, L), jnp.float32),
        pltpu.VMEM((HR, L), jnp.float32),
        pltpu.VMEM((HR, L), jnp.float32),
        pltpu.VMEM((EPT,), jnp.int32),
        pltpu.VMEM((EPT,), jnp.int32),
        pltpu.VMEM((HR,), jnp.int32),
        pltpu.VMEM((RPT // L, L), jnp.float32),
    ],
    compiler_params=_SC_PARAMS,
)
def _degree_norms(src_hbm, dst_hbm, ns_out, nd_out, *scratch):
    _degree_norm_body(src_hbm, dst_hbm, ns_out, nd_out, *scratch)


def _agg_body(xs_hbm, src_hbm, dst_hbm, ew_hbm, out_hbm,
              acc, ebuf_s, ebuf_d, ebuf_w, rows_a, rows_b, gdst_a, gdst_b,
              gsem_a, gsem_b, ssem_a, ssem_b):
    c = lax.axis_index("c")
    s = lax.axis_index("s")
    off = c * HALF
    zero16 = jnp.zeros((L,), jnp.float32)
    dummy16 = jnp.full((L,), DUMMY, jnp.int32)
    zero16i = jnp.zeros((L,), jnp.int32)

    def zrow(r, _):
        for j in range(D // L):
            rows_a[r, pl.ds(j * L, L)] = zero16
        return 0
    lax.fori_loop(0, K, zrow, 0)

    for i in range(ZPT // K):
        pltpu.sync_copy(rows_a, acc.at[pl.ds(s * ZPT + i * K, K)])
    if ZPT % K:
        pltpu.sync_copy(rows_a.at[pl.ds(0, ZPT % K)],
                        acc.at[pl.ds(s * ZPT + (ZPT // K) * K, ZPT % K)])
    plsc.subcore_barrier()

    rows = (rows_a, rows_b)
    gdst = (gdst_a, gdst_b)
    gsem = (gsem_a, gsem_b)
    ssem = (ssem_a, ssem_b)

    eoff = 0
    for r in range(2):
        ngrp = RGRP[r]
        ept_r = ngrp * L
        pltpu.sync_copy(src_hbm.at[pl.ds(s * EPT + eoff, ept_r)],
                        ebuf_s.at[pl.ds(0, ept_r)])
        pltpu.sync_copy(dst_hbm.at[pl.ds(s * EPT + eoff, ept_r)],
                        ebuf_d.at[pl.ds(0, ept_r)])
        pltpu.sync_copy(ew_hbm.at[pl.ds(s * EPT + eoff, ept_r)],
                        ebuf_w.at[pl.ds(0, ept_r)])
        eoff += ept_r

        def scan(g, cnt):
            sv = ebuf_s[pl.ds(g * L, L)]
            dv = ebuf_d[pl.ds(g * L, L)]
            wv = ebuf_w[pl.ds(g * L, L)]
            ld = dv - off
            m = (ld >= 0) & (ld < HALF)
            plsc.store_compressed(ebuf_s.at[pl.ds(cnt, L)], sv, mask=m)
            plsc.store_compressed(ebuf_d.at[pl.ds(cnt, L)], ld, mask=m)
            plsc.store_compressed(ebuf_w.at[pl.ds(cnt, L)], wv, mask=m)
            return cnt + jnp.max(plsc.all_reduce_population_count(m))
        cnt = lax.fori_loop(0, ngrp, scan, jnp.int32(0))

        def tail(t, _):
            ebuf_d[pl.ds(cnt + t * L, L)] = dummy16
            ebuf_s[pl.ds(cnt + t * L, L)] = zero16i
            return 0
        lax.fori_loop(0, 2 * K // L, tail, 0)

        npair = (cnt + 2 * K - 1) // (2 * K)
        npair = jnp.maximum(npair, 1)
        nch = npair * 2

        def issue_gather(b, ci):
            cc = jnp.minimum(ci, nch - 1)
            pltpu.async_copy(xs_hbm.at[ebuf_s.at[pl.ds(cc * K, K)]],
                             rows[b], gsem[b])

        def wait_gather(b):
            pltpu.make_async_copy(xs_hbm.at[ebuf_s.at[pl.ds(0, K)]],
                                  rows[b], gsem[b]).wait()

        def process(b, ci):
            base = ci * K

            def cpy(t, _):
                gdst[b][pl.ds(t * L, L)] = ebuf_d[pl.ds(base + t * L, L)]
                return 0
            lax.fori_loop(0, K // L, cpy, 0)

            def mul(e, _):
                w = plsc.load_gather(
                    ebuf_w, [jnp.full((L,), base, jnp.int32) + e])
                for j in range(D // L):
                    rows[b][e, pl.ds(j * L, L)] = \
                        rows[b][e, pl.ds(j * L, L)] * w
                return 0
            lax.fori_loop(0, K, mul, 0)

            pltpu.async_copy(rows[b], acc.at[gdst[b]], ssem[b], add=True)

        def wait_scatter(b):
            pltpu.make_async_copy(rows[b], acc.at[gdst[b]], ssem[b]).wait()

        issue_gather(0, jnp.int32(0))
        issue_gather(1, jnp.int32(1))

        def pipe(p, _):
            wait_gather(0)
            process(0, 2 * p)
            wait_gather(1)
            process(1, 2 * p + 1)
            wait_scatter(0)
            issue_gather(0, 2 * p + 2)
            wait_scatter(1)
            issue_gather(1, 2 * p + 3)
            return 0
        lax.fori_loop(0, npair, pipe, 0)

        wait_gather(0)
        wait_gather(1)

    plsc.subcore_barrier()
    pltpu.sync_copy(acc.at[pl.ds(s * RPT, RPT)],
                    out_hbm.at[pl.ds(off + s * RPT, RPT)])


@functools.partial(
    pl.kernel,
    out_type=jax.ShapeDtypeStruct((NPAD, D), jnp.float32),
    mesh=_MESH,
    scratch_types=[
        pltpu.VMEM_SHARED((ACC_ROWS, D), jnp.float32),
        pltpu.VMEM((EBUF,), jnp.int32),
        pltpu.VMEM((EBUF,), jnp.int32),
        pltpu.VMEM((EBUF,), jnp.float32),
        pltpu.VMEM((K, D), jnp.float32),
        pltpu.VMEM((K, D), jnp.float32),
        pltpu.VMEM((K,), jnp.int32),
        pltpu.VMEM((K,), jnp.int32),
        pltpu.SemaphoreType.DMA,
        pltpu.SemaphoreType.DMA,
        pltpu.SemaphoreType.DMA,
        pltpu.SemaphoreType.DMA,
    ],
    compiler_params=_SC_PARAMS,
)
def _agg(xs_hbm, src_hbm, dst_hbm, ew_hbm, out_hbm, *scratch):
    _agg_body(xs_hbm, src_hbm, dst_hbm, ew_hbm, out_hbm, *scratch)


def _dense_kernel(nd_ref, a_ref, w_ref, b_ref, o_ref):
    a = a_ref[...] * nd_ref[...]
    acc = jnp.dot(a, w_ref[...], preferred_element_type=jnp.float32)
    o_ref[...] = jnp.maximum(acc + b_ref[...], 0.0)


def _dense_scaled_kernel(nd_ref, a_ref, w_ref, b_ref, ns_ref, o_ref):
    a = a_ref[...] * nd_ref[...]
    acc = jnp.dot(a, w_ref[...], preferred_element_type=jnp.float32)
    o_ref[...] = jnp.maximum(acc + b_ref[...], 0.0) * ns_ref[...]


_BN = 1000


def _dense(agg, nd, W, b, ns=None):
    col = pl.BlockSpec((_BN, 1), lambda i: (i, 0))
    specs = [
        col,
        pl.BlockSpec((_BN, D), lambda i: (i, 0)),
        pl.BlockSpec((D, D), lambda i: (0, 0)),
        pl.BlockSpec((1, D), lambda i: (0, 0)),
    ]
    args = [nd, agg, W, b.reshape(1, D)]
    body = _dense_kernel
    if ns is not None:
        specs.append(col)
        args.append(ns)
        body = _dense_scaled_kernel
    return pl.pallas_call(
        body,
        grid=(N // _BN,),
        in_specs=specs,
        out_specs=pl.BlockSpec((_BN, D), lambda i: (i, 0)),
        out_shape=jax.ShapeDtypeStruct((N, D), jnp.float32),
    )(*args)


def _rowscale_kernel(x_ref, ns_ref, o_ref):
    o_ref[...] = x_ref[...] * ns_ref[...]


def _rowscale(x, ns):
    return pl.pallas_call(
        _rowscale_kernel,
        grid=(N // _BN,),
        in_specs=[pl.BlockSpec((_BN, D), lambda i: (i, 0)),
                  pl.BlockSpec((_BN, 1), lambda i: (i, 0))],
        out_specs=pl.BlockSpec((_BN, D), lambda i: (i, 0)),
        out_shape=jax.ShapeDtypeStruct((N, D), jnp.float32),
    )(x, ns)


def kernel(x, edge_index, edge_weight, W1, b1, W2, b2):
    src = edge_index[0]
    dst = edge_index[1]
    ns, nd = _degree_norms(src, dst)
    ns2 = ns.reshape(NPAD)[:N].reshape(N, 1)
    nd2 = nd.reshape(NPAD)[:N].reshape(N, 1)
    xs = _rowscale(x, ns2)
    agg1 = _agg(xs, src, dst, edge_weight)[:N]
    h1s = _dense(agg1, nd2, W1, b1, ns=ns2)
    agg2 = _agg(h1s, src, dst, edge_weight)[:N]
    return _dense(agg2, nd2, W2, b2)
